# baseline (speedup 1.0000x reference)
import functools

import jax
import numpy as np
import jax.numpy as jnp
from jax import lax
from jax.experimental import pallas as pl
from jax.experimental.pallas import tpu as pltpu
from jax.experimental.pallas import tpu_sc as plsc

USER_SIZE = 100000
ITEM_SIZE = 100000
EMBED = 64
EPAD = 128
B = 4096
A = 0.2

_INFO = plsc.get_sparse_core_info()
NC = _INFO.num_cores
NS = _INFO.num_subcores
NW = NC * NS
BH = B // 2
CHUNK = BH // NW

_TCOLS = 20480
_TSTEPS = -(-USER_SIZE // _TCOLS)

_DN = (((0,), (0,)), ((), ()))

_E1 = np.eye(EMBED, EPAD, dtype=np.float32)
_E2 = np.eye(EMBED, EPAD, k=EMBED, dtype=np.float32)
_OMAT = (np.arange(5 * EMBED)[:, None] // EMBED
         == np.arange(EPAD)[None, :] // 8).astype(np.float32)
_QMAT = (np.arange(40)[:, None] // 8
         == np.arange(5 * EMBED)[None, :] // EMBED).astype(np.float32) / 8.0


def _repack_kernel(wu_ref, wi_ref, e1_ref, e2_ref, brow_ref, out_ref):
    t = lax.dot_general(wu_ref[...], e1_ref[...], _DN,
                        preferred_element_type=jnp.float32)
    t = t + lax.dot_general(wi_ref[...], e2_ref[...], _DN,
                            preferred_element_type=jnp.float32)
    out_ref[...] = jnp.maximum(t + brow_ref[...], 0.0)


_repack = pl.pallas_call(
    _repack_kernel,
    grid=(_TSTEPS,),
    in_specs=[
        pl.BlockSpec((EMBED, _TCOLS), lambda w: (0, w)),
        pl.BlockSpec((EMBED, _TCOLS), lambda w: (0, w)),
        pl.BlockSpec((EMBED, EPAD), lambda w: (0, 0)),
        pl.BlockSpec((EMBED, EPAD), lambda w: (0, 0)),
        pl.BlockSpec((1, EPAD), lambda w: (0, 0)),
    ],
    out_specs=pl.BlockSpec((_TCOLS, EPAD), lambda w: (w, 0)),
    out_shape=jax.ShapeDtypeStruct((USER_SIZE, EPAD), jnp.float32),
)


def _build_gather(half):
    mesh = plsc.VectorSubcoreMesh(core_axis_name="c", subcore_axis_name="s")

    @functools.partial(
        pl.kernel,
        mesh=mesh,
        compiler_params=pltpu.CompilerParams(use_tc_tiling_on_sc=True),
        out_type=jax.ShapeDtypeStruct((7, BH, EPAD), jnp.float32),
        scratch_types=[
            pltpu.VMEM((7, CHUNK), jnp.int32),
        ] + [pltpu.VMEM((CHUNK, EPAD), jnp.float32) for _ in range(7)] + [
            pltpu.SemaphoreType.DMA,
            pltpu.SemaphoreType.DMA,
            pltpu.SemaphoreType.DMA,
        ],
    )
    def gather_kernel(tbl_hbm, u_hbm, i_hbm, p1_hbm, p2_hbm, p3_hbm,
                      p4_hbm, p5_hbm, out_hbm, idx_all, v0, v1, v2, v3, v4,
                      v5, v6, isem, gsem, wsem):
        wid = lax.axis_index("s") * NC + lax.axis_index("c")
        base = half * BH + wid * CHUNK
        idx_hbms = [u_hbm, i_hbm, p1_hbm, p2_hbm, p3_hbm, p4_hbm, p5_hbm]
        hs = [pltpu.async_copy(ih.at[pl.ds(base, CHUNK)], idx_all.at[v], isem)
              for v, ih in enumerate(idx_hbms)]
        for h in hs:
            h.wait()
        vals = [v0, v1, v2, v3, v4, v5, v6]
        g = [pltpu.async_copy(tbl_hbm.at[idx_all.at[v]], vals[v], gsem)
             for v in range(7)]
        w = [None] * 7
        for v in range(7):
            g[v].wait()
            w[v] = pltpu.async_copy(vals[v],
                                    out_hbm.at[v].at[pl.ds(base, CHUNK)], wsem)
        for v in range(7):
            w[v].wait()

    return gather_kernel


_gather0 = _build_gather(0)
_gather1 = _build_gather(1)


_DW = (((1,), (1,)), ((), ()))


def _dense_kernel(g_ref, o_ref, q_ref, w1_ref, b1_ref, w2_ref, b2_ref,
                  w3_ref, b3_ref, out_ref):
    eu = g_ref[0][:, :EMBED]
    ei = g_ref[1][:, EMBED:]
    es = [g_ref[k][:, EMBED:] for k in range(2, 7)]
    p = jnp.concatenate([ei * e for e in es], axis=1)
    r = jnp.dot(p, o_ref[...], preferred_element_type=jnp.float32)
    x = jnp.exp(r)
    s = (x[:, 0:8] + x[:, 8:16] + x[:, 16:24] + x[:, 24:32]
         + x[:, 32:40])
    z8 = x[:, :40] / jnp.tile(s, (1, 5))
    zf = jnp.dot(z8, q_ref[...], preferred_element_type=jnp.float32)
    pum = sum(zf[:, 64 * k:64 * (k + 1)] * e
              for k, e in enumerate(es))
    pu = eu + A * pum
    x1 = jnp.concatenate([pu, ei], axis=1)
    h = jnp.maximum(
        jnp.dot(x1, w1_ref[...], preferred_element_type=jnp.float32)
        + b1_ref[...], 0.0)
    h = jnp.maximum(
        jnp.dot(h, w2_ref[...], preferred_element_type=jnp.float32)
        + b2_ref[...], 0.0)
    o = (jnp.dot(h, w3_ref[...], preferred_element_type=jnp.float32)
         + b3_ref[...])
    out_ref[...] = 1.0 / (1.0 + jnp.exp(-o))


_dense = pl.pallas_call(
    _dense_kernel,
    out_shape=jax.ShapeDtypeStruct((BH, 1), jnp.float32),
)


def kernel(Wu, bu, Wi, bi, W1, b1, W2, b2, W3, b3,
           user, item, pre1, pre2, pre3, pre4, pre5):
    i32 = jnp.int32
    packed = _repack(Wu, Wi, _E1, _E2,
                     jnp.concatenate([bu, bi]).reshape(1, EPAD))
    idx = [a.astype(i32) for a in (user, item, pre1, pre2, pre3, pre4, pre5)]
    preds = []
    for gather_h in (_gather0, _gather1):
        gath = gather_h(packed, *idx)
        preds.append(_dense(gath, _OMAT, _QMAT, W1.T,
                            b1.reshape(1, 2 * EMBED), W2.T,
                            b2.reshape(1, EMBED), W3.T, b3.reshape(1, 1)))
    return jnp.concatenate(preds).reshape(-1)


# speedup vs baseline: 1.0005x; 1.0005x over previous
import functools

import jax
import numpy as np
import jax.numpy as jnp
from jax import lax
from jax.experimental import pallas as pl
from jax.experimental.pallas import tpu as pltpu
from jax.experimental.pallas import tpu_sc as plsc

USER_SIZE = 100000
ITEM_SIZE = 100000
EMBED = 64
EPAD = 128
B = 4096
A = 0.2

_INFO = plsc.get_sparse_core_info()
NC = _INFO.num_cores
NS = _INFO.num_subcores
NW = NC * NS
BH = B // 2
CHUNK = BH // NW

_TCOLS = 20480
_TSTEPS = -(-USER_SIZE // _TCOLS)

_DN = (((0,), (0,)), ((), ()))

_E1 = np.eye(EMBED, EPAD, dtype=np.float32)
_E2 = np.eye(EMBED, EPAD, k=EMBED, dtype=np.float32)
_OMAT = (np.arange(5 * EMBED)[:, None] // EMBED
         == np.arange(EPAD)[None, :] // 8).astype(np.float32)
_QMAT = (np.arange(40)[:, None] // 8
         == np.arange(5 * EMBED)[None, :] // EMBED).astype(np.float32) / 8.0


def _repack_kernel(wu_ref, wi_ref, e1_ref, e2_ref, brow_ref, out_ref):
    t = lax.dot_general(wu_ref[...], e1_ref[...], _DN,
                        preferred_element_type=jnp.float32)
    t = t + lax.dot_general(wi_ref[...], e2_ref[...], _DN,
                            preferred_element_type=jnp.float32)
    out_ref[...] = jnp.maximum(t + brow_ref[...], 0.0)


_repack = pl.pallas_call(
    _repack_kernel,
    grid=(_TSTEPS,),
    in_specs=[
        pl.BlockSpec((EMBED, _TCOLS), lambda w: (0, w)),
        pl.BlockSpec((EMBED, _TCOLS), lambda w: (0, w)),
        pl.BlockSpec((EMBED, EPAD), lambda w: (0, 0)),
        pl.BlockSpec((EMBED, EPAD), lambda w: (0, 0)),
        pl.BlockSpec((1, EPAD), lambda w: (0, 0)),
    ],
    out_specs=pl.BlockSpec((_TCOLS, EPAD), lambda w: (w, 0)),
    out_shape=jax.ShapeDtypeStruct((USER_SIZE, EPAD), jnp.float32),
)


def _build_gather(half):
    mesh = plsc.VectorSubcoreMesh(core_axis_name="c", subcore_axis_name="s")

    @functools.partial(
        pl.kernel,
        mesh=mesh,
        compiler_params=pltpu.CompilerParams(use_tc_tiling_on_sc=True),
        out_type=jax.ShapeDtypeStruct((7, BH, EPAD), jnp.float32),
        scratch_types=[
            pltpu.VMEM((7, CHUNK), jnp.int32),
        ] + [pltpu.VMEM((CHUNK, EPAD), jnp.float32) for _ in range(7)] + [
            pltpu.SemaphoreType.DMA,
            pltpu.SemaphoreType.DMA,
            pltpu.SemaphoreType.DMA,
        ],
    )
    def gather_kernel(tbl_hbm, u_hbm, i_hbm, p1_hbm, p2_hbm, p3_hbm,
                      p4_hbm, p5_hbm, out_hbm, idx_all, v0, v1, v2, v3, v4,
                      v5, v6, isem, gsem, wsem):
        wid = lax.axis_index("s") * NC + lax.axis_index("c")
        base = wid * CHUNK
        gbase = half * BH + wid * CHUNK
        idx_hbms = [u_hbm, i_hbm, p1_hbm, p2_hbm, p3_hbm, p4_hbm, p5_hbm]
        hs = [pltpu.async_copy(ih.at[pl.ds(gbase, CHUNK)], idx_all.at[v], isem)
              for v, ih in enumerate(idx_hbms)]
        for h in hs:
            h.wait()
        vals = [v0, v1, v2, v3, v4, v5, v6]
        g = [pltpu.async_copy(tbl_hbm.at[idx_all.at[v]], vals[v], gsem)
             for v in range(7)]
        w = [None] * 7
        for v in range(7):
            g[v].wait()
            w[v] = pltpu.async_copy(vals[v],
                                    out_hbm.at[v].at[pl.ds(base, CHUNK)], wsem)
        for v in range(7):
            w[v].wait()

    return gather_kernel


_gather0 = _build_gather(0)
_gather1 = _build_gather(1)


_DW = (((1,), (1,)), ((), ()))


def _dense_kernel(g_ref, o_ref, q_ref, w1_ref, b1_ref, w2_ref, b2_ref,
                  w3_ref, b3_ref, out_ref):
    eu = g_ref[0][:, :EMBED]
    ei = g_ref[1][:, EMBED:]
    es = [g_ref[k][:, EMBED:] for k in range(2, 7)]
    p = jnp.concatenate([ei * e for e in es], axis=1)
    r = jnp.dot(p, o_ref[...], preferred_element_type=jnp.float32)
    x = jnp.exp(r)
    s = (x[:, 0:8] + x[:, 8:16] + x[:, 16:24] + x[:, 24:32]
         + x[:, 32:40])
    z8 = x[:, :40] / jnp.tile(s, (1, 5))
    zf = jnp.dot(z8, q_ref[...], preferred_element_type=jnp.float32)
    pum = sum(zf[:, 64 * k:64 * (k + 1)] * e
              for k, e in enumerate(es))
    pu = eu + A * pum
    x1 = jnp.concatenate([pu, ei], axis=1)
    h = jnp.maximum(
        jnp.dot(x1, w1_ref[...], preferred_element_type=jnp.float32)
        + b1_ref[...], 0.0)
    h = jnp.maximum(
        jnp.dot(h, w2_ref[...], preferred_element_type=jnp.float32)
        + b2_ref[...], 0.0)
    o = (jnp.dot(h, w3_ref[...], preferred_element_type=jnp.float32)
         + b3_ref[...])
    out_ref[...] = 1.0 / (1.0 + jnp.exp(-o))


_dense = pl.pallas_call(
    _dense_kernel,
    out_shape=jax.ShapeDtypeStruct((BH, 1), jnp.float32),
)


def kernel(Wu, bu, Wi, bi, W1, b1, W2, b2, W3, b3,
           user, item, pre1, pre2, pre3, pre4, pre5):
    i32 = jnp.int32
    packed = _repack(Wu, Wi, _E1, _E2,
                     jnp.concatenate([bu, bi]).reshape(1, EPAD))
    idx = [a.astype(i32) for a in (user, item, pre1, pre2, pre3, pre4, pre5)]
    preds = []
    for gather_h in (_gather0, _gather1):
        gath = gather_h(packed, *idx)
        preds.append(_dense(gath, _OMAT, _QMAT, W1.T,
                            b1.reshape(1, 2 * EMBED), W2.T,
                            b2.reshape(1, EMBED), W3.T, b3.reshape(1, 1)))
    return jnp.concatenate(preds).reshape(-1)


# speedup vs baseline: 1.0089x; 1.0084x over previous
import functools

import jax
import numpy as np
import jax.numpy as jnp
from jax import lax
from jax.experimental import pallas as pl
from jax.experimental.pallas import tpu as pltpu
from jax.experimental.pallas import tpu_sc as plsc

USER_SIZE = 100000
ITEM_SIZE = 100000
EMBED = 64
EPAD = 128
B = 4096
A = 0.2

_INFO = plsc.get_sparse_core_info()
NC = _INFO.num_cores
NS = _INFO.num_subcores
NW = NC * NS
BH = B // 2
CHUNK = BH // NW

_TCOLS = 25088
_TSTEPS = -(-USER_SIZE // _TCOLS)

_DN = (((0,), (0,)), ((), ()))

_E1 = np.eye(EMBED, EPAD, dtype=np.float32)
_E2 = np.eye(EMBED, EPAD, k=EMBED, dtype=np.float32)
_OMAT = (np.arange(5 * EMBED)[:, None] // EMBED
         == np.arange(EPAD)[None, :] // 8).astype(np.float32)
_QMAT = (np.arange(40)[:, None] // 8
         == np.arange(5 * EMBED)[None, :] // EMBED).astype(np.float32) / 8.0


def _repack_kernel(wu_ref, wi_ref, e1_ref, e2_ref, brow_ref, out_ref):
    t = lax.dot_general(wu_ref[...], e1_ref[...], _DN,
                        preferred_element_type=jnp.float32)
    t = t + lax.dot_general(wi_ref[...], e2_ref[...], _DN,
                            preferred_element_type=jnp.float32)
    out_ref[...] = jnp.maximum(t + brow_ref[...], 0.0)


_repack = pl.pallas_call(
    _repack_kernel,
    grid=(_TSTEPS,),
    compiler_params=pltpu.CompilerParams(vmem_limit_bytes=112 * 1024 * 1024),
    in_specs=[
        pl.BlockSpec((EMBED, _TCOLS), lambda w: (0, w)),
        pl.BlockSpec((EMBED, _TCOLS), lambda w: (0, w)),
        pl.BlockSpec((EMBED, EPAD), lambda w: (0, 0)),
        pl.BlockSpec((EMBED, EPAD), lambda w: (0, 0)),
        pl.BlockSpec((1, EPAD), lambda w: (0, 0)),
    ],
    out_specs=pl.BlockSpec((_TCOLS, EPAD), lambda w: (w, 0)),
    out_shape=jax.ShapeDtypeStruct((USER_SIZE, EPAD), jnp.float32),
)


def _build_gather(half):
    mesh = plsc.VectorSubcoreMesh(core_axis_name="c", subcore_axis_name="s")

    @functools.partial(
        pl.kernel,
        mesh=mesh,
        compiler_params=pltpu.CompilerParams(use_tc_tiling_on_sc=True),
        out_type=jax.ShapeDtypeStruct((7, BH, EPAD), jnp.float32),
        scratch_types=[
            pltpu.VMEM((7, CHUNK), jnp.int32),
        ] + [pltpu.VMEM((CHUNK, EPAD), jnp.float32) for _ in range(7)] + [
            pltpu.SemaphoreType.DMA,
            pltpu.SemaphoreType.DMA,
            pltpu.SemaphoreType.DMA,
        ],
    )
    def gather_kernel(tbl_hbm, u_hbm, i_hbm, p1_hbm, p2_hbm, p3_hbm,
                      p4_hbm, p5_hbm, out_hbm, idx_all, v0, v1, v2, v3, v4,
                      v5, v6, isem, gsem, wsem):
        wid = lax.axis_index("s") * NC + lax.axis_index("c")
        base = wid * CHUNK
        gbase = half * BH + wid * CHUNK
        idx_hbms = [u_hbm, i_hbm, p1_hbm, p2_hbm, p3_hbm, p4_hbm, p5_hbm]
        hs = [pltpu.async_copy(ih.at[pl.ds(gbase, CHUNK)], idx_all.at[v], isem)
              for v, ih in enumerate(idx_hbms)]
        for h in hs:
            h.wait()
        vals = [v0, v1, v2, v3, v4, v5, v6]
        g = [pltpu.async_copy(tbl_hbm.at[idx_all.at[v]], vals[v], gsem)
             for v in range(7)]
        w = [None] * 7
        for v in range(7):
            g[v].wait()
            w[v] = pltpu.async_copy(vals[v],
                                    out_hbm.at[v].at[pl.ds(base, CHUNK)], wsem)
        for v in range(7):
            w[v].wait()

    return gather_kernel


_gather0 = _build_gather(0)
_gather1 = _build_gather(1)


_DW = (((1,), (1,)), ((), ()))


def _dense_kernel(g_ref, o_ref, q_ref, w1_ref, b1_ref, w2_ref, b2_ref,
                  w3_ref, b3_ref, out_ref):
    eu = g_ref[0][:, :EMBED]
    ei = g_ref[1][:, EMBED:]
    es = [g_ref[k][:, EMBED:] for k in range(2, 7)]
    p = jnp.concatenate([ei * e for e in es], axis=1)
    r = jnp.dot(p, o_ref[...], preferred_element_type=jnp.float32)
    x = jnp.exp(r)
    s = (x[:, 0:8] + x[:, 8:16] + x[:, 16:24] + x[:, 24:32]
         + x[:, 32:40])
    z8 = x[:, :40] / jnp.tile(s, (1, 5))
    zf = jnp.dot(z8, q_ref[...], preferred_element_type=jnp.float32)
    pum = sum(zf[:, 64 * k:64 * (k + 1)] * e
              for k, e in enumerate(es))
    pu = eu + A * pum
    x1 = jnp.concatenate([pu, ei], axis=1)
    h = jnp.maximum(
        jnp.dot(x1, w1_ref[...], preferred_element_type=jnp.float32)
        + b1_ref[...], 0.0)
    h = jnp.maximum(
        jnp.dot(h, w2_ref[...], preferred_element_type=jnp.float32)
        + b2_ref[...], 0.0)
    o = (jnp.dot(h, w3_ref[...], preferred_element_type=jnp.float32)
         + b3_ref[...])
    out_ref[...] = 1.0 / (1.0 + jnp.exp(-o))


_dense = pl.pallas_call(
    _dense_kernel,
    out_shape=jax.ShapeDtypeStruct((BH, 1), jnp.float32),
)


def kernel(Wu, bu, Wi, bi, W1, b1, W2, b2, W3, b3,
           user, item, pre1, pre2, pre3, pre4, pre5):
    i32 = jnp.int32
    packed = _repack(Wu, Wi, _E1, _E2,
                     jnp.concatenate([bu, bi]).reshape(1, EPAD))
    idx = [a.astype(i32) for a in (user, item, pre1, pre2, pre3, pre4, pre5)]
    preds = []
    for gather_h in (_gather0, _gather1):
        gath = gather_h(packed, *idx)
        preds.append(_dense(gath, _OMAT, _QMAT, W1.T,
                            b1.reshape(1, 2 * EMBED), W2.T,
                            b2.reshape(1, EMBED), W3.T, b3.reshape(1, 1)))
    return jnp.concatenate(preds).reshape(-1)
